# baseline (speedup 1.0000x reference)
import functools
import math

import jax
import jax.numpy as jnp
from jax.experimental import pallas as pl
from jax.experimental.pallas import tpu as pltpu

TOKEN_BLOCK = 512


def _moe_kernel(x_ref, wg_ref, bg_ref, we_ref, be_ref, out_ref, we16_ref,
                *, n_experts):
    @pl.when(pl.program_id(0) == 0)
    def _cast_weights():
        we16_ref[...] = we_ref[...].astype(jnp.bfloat16)

    x = x_ref[...]
    x16 = x.astype(jnp.bfloat16)
    logits = jnp.dot(x, wg_ref[...], preferred_element_type=jnp.float32)
    logits = logits + bg_ref[...]
    logits = logits - jnp.max(logits, axis=1, keepdims=True)
    p = jnp.exp(logits)
    w2 = p * (0.5 / jnp.sum(p, axis=1, keepdims=True))

    acc = jnp.zeros(out_ref.shape, dtype=jnp.float32)
    for e in range(n_experts):
        h = jnp.dot(x16, we16_ref[e], preferred_element_type=jnp.float32)
        h = h + be_ref[e]
        wh = w2[:, e:e + 1] * h
        acc = acc + (wh + wh * jax.lax.erf(h * (1.0 / math.sqrt(2.0))))
    out_ref[...] = acc


def kernel(x, Wg, bg, We, be):
    T, D_IN = x.shape
    E = We.shape[0]
    D_OUT = We.shape[2]
    bt = min(TOKEN_BLOCK, T)
    grid = (T // bt,)

    return pl.pallas_call(
        functools.partial(_moe_kernel, n_experts=E),
        grid=grid,
        in_specs=[
            pl.BlockSpec((bt, D_IN), lambda i: (i, 0)),
            pl.BlockSpec((D_IN, E), lambda i: (0, 0)),
            pl.BlockSpec((1, E), lambda i: (0, 0)),
            pl.BlockSpec((E, D_IN, D_OUT), lambda i: (0, 0, 0)),
            pl.BlockSpec((E, D_OUT), lambda i: (0, 0)),
        ],
        out_specs=pl.BlockSpec((bt, D_OUT), lambda i: (i, 0)),
        out_shape=jax.ShapeDtypeStruct((T, D_OUT), jnp.float32),
        scratch_shapes=[pltpu.VMEM((E, D_IN, D_OUT), jnp.bfloat16)],
        compiler_params=pltpu.CompilerParams(
            vmem_limit_bytes=100 * 1024 * 1024,
        ),
    )(x, Wg, bg.reshape(1, E), We, be)


# speedup vs baseline: 1.0493x; 1.0493x over previous
import functools
import math

import jax
import jax.numpy as jnp
from jax.experimental import pallas as pl
from jax.experimental.pallas import tpu as pltpu

TOKEN_BLOCK = 1024


def _moe_kernel(x_ref, wg_ref, bg_ref, we_ref, be_ref, out_ref, we16_ref,
                h_ref, *, n_experts):
    @pl.when(pl.program_id(0) == 0)
    def _cast_weights():
        we16_ref[...] = we_ref[...].astype(jnp.bfloat16)

    x = x_ref[...]
    x16 = x.astype(jnp.bfloat16)
    logits = jnp.dot(x, wg_ref[...], preferred_element_type=jnp.float32)
    logits = logits + bg_ref[...]
    logits = logits - jnp.max(logits, axis=1, keepdims=True)
    p = jnp.exp(logits)
    w2 = p * (0.5 / jnp.sum(p, axis=1, keepdims=True))

    for e in range(n_experts):
        h_ref[e % 2] = jnp.dot(x16, we16_ref[e],
                               preferred_element_type=jnp.float32)
        h = h_ref[e % 2] + be_ref[e]
        wh = w2[:, e:e + 1] * h
        g = wh + wh * jax.lax.erf(h * (1.0 / math.sqrt(2.0)))
        if e == 0:
            out_ref[...] = g
        else:
            out_ref[...] += g


def kernel(x, Wg, bg, We, be):
    T, D_IN = x.shape
    E = We.shape[0]
    D_OUT = We.shape[2]
    bt = min(TOKEN_BLOCK, T)
    grid = (T // bt,)

    return pl.pallas_call(
        functools.partial(_moe_kernel, n_experts=E),
        grid=grid,
        in_specs=[
            pl.BlockSpec((bt, D_IN), lambda i: (i, 0)),
            pl.BlockSpec((D_IN, E), lambda i: (0, 0)),
            pl.BlockSpec((1, E), lambda i: (0, 0)),
            pl.BlockSpec((E, D_IN, D_OUT), lambda i: (0, 0, 0)),
            pl.BlockSpec((E, D_OUT), lambda i: (0, 0)),
        ],
        out_specs=pl.BlockSpec((bt, D_OUT), lambda i: (i, 0)),
        out_shape=jax.ShapeDtypeStruct((T, D_OUT), jnp.float32),
        scratch_shapes=[
            pltpu.VMEM((E, D_IN, D_OUT), jnp.bfloat16),
            pltpu.VMEM((2, bt, D_OUT), jnp.float32),
        ],
        compiler_params=pltpu.CompilerParams(
            vmem_limit_bytes=100 * 1024 * 1024,
        ),
    )(x, Wg, bg.reshape(1, E), We, be)
